# Initial kernel scaffold; baseline (speedup 1.0000x reference)
import jax
import jax.numpy as jnp
from jax.experimental import pallas as pl


def kernel(src, tgt, noise):
    raise NotImplementedError("write your pallas kernel here")


# speedup vs baseline: 31.1934x; 31.1934x over previous
import functools

import jax
import jax.numpy as jnp
from jax.experimental import pallas as pl


_K = 5
_BETA = 3.0


def _body(q_ref, xpt_t_ref, xp_t_ref, xpt_s_ref, xp_s_ref, out_ref):
    q = q_ref[0, 0]
    qq = jnp.sum(q * q, axis=1, keepdims=True)

    res = []
    for xpt_ref, xp_ref in ((xpt_t_ref, xp_t_ref), (xpt_s_ref, xp_s_ref)):
        xt = xpt_ref[0]
        xp = xp_ref[0]
        xx = jnp.sum(xt * xt, axis=0, keepdims=True)
        d2 = qq + xx - 2.0 * jnp.dot(q, xt, preferred_element_type=jnp.float32)
        d2 = jnp.maximum(d2, 0.0)
        iota = jax.lax.broadcasted_iota(jnp.int32, d2.shape, 1)
        w_mat = jnp.zeros_like(d2)
        norm = jnp.zeros_like(qq)
        for _ in range(_K):
            m = jnp.min(d2, axis=1, keepdims=True)
            msk = d2 == m
            li = jnp.min(
                jnp.where(msk, iota, jnp.int32(2**30)), axis=1, keepdims=True
            )
            sel = iota == li
            w = 1.0 / (m + 1e-8)
            w_mat = w_mat + jnp.where(sel, w, 0.0)
            norm = norm + w
            d2 = jnp.where(sel, jnp.float32(1e30), d2)
        p = jnp.dot(w_mat, xp, preferred_element_type=jnp.float32)
        g = q - p / norm
        udf = jnp.sqrt(jnp.sum((g + 1e-10) ** 2, axis=1, keepdims=True))
        res.append((udf, g))

    (udf_t, g_t), (udf_s, g_s) = res
    ue = jnp.abs(udf_t - udf_s)
    ge = jnp.sum(jnp.abs(g_s - g_t), axis=1, keepdims=True)
    tot = ue + ge
    out_ref[0, 0] = tot * jnp.exp(-tot * _BETA)


@functools.partial(jax.jit, static_argnames=("interpret",))
def _impl(src, tgt, noise, interpret=False):
    b, n_tgt, _ = tgt.shape
    n_src = src.shape[1]
    up = noise.shape[2]
    nq = n_tgt * up + n_src

    query = tgt[:, :, None, :] + noise
    query = query.reshape(b, n_tgt * up, 3)
    query = jnp.concatenate([query, src], axis=1)

    qblk = 256
    assert nq % qblk == 0
    nb = nq // qblk

    def pad8(a):
        return jnp.pad(a, ((0, 0), (0, 0), (0, 5)))

    qp = pad8(query).reshape(b, nb, qblk, 8)
    xp_t = pad8(tgt)
    xp_s = pad8(src)
    xpt_t = xp_t.transpose(0, 2, 1)
    xpt_s = xp_s.transpose(0, 2, 1)

    r_t = n_tgt
    r_s = n_src

    contrib = pl.pallas_call(
        _body,
        grid=(b, nb),
        in_specs=[
            pl.BlockSpec((1, 1, qblk, 8), lambda bi, i: (bi, i, 0, 0)),
            pl.BlockSpec((1, 8, r_t), lambda bi, i: (bi, 0, 0)),
            pl.BlockSpec((1, r_t, 8), lambda bi, i: (bi, 0, 0)),
            pl.BlockSpec((1, 8, r_s), lambda bi, i: (bi, 0, 0)),
            pl.BlockSpec((1, r_s, 8), lambda bi, i: (bi, 0, 0)),
        ],
        out_specs=pl.BlockSpec((1, 1, qblk, 1), lambda bi, i: (bi, i, 0, 0)),
        out_shape=jax.ShapeDtypeStruct((b, nb, qblk, 1), jnp.float32),
        interpret=interpret,
    )(qp, xpt_t, xp_t, xpt_s, xp_s)

    return jnp.sum(contrib) / b / nq


def kernel(src, tgt, noise):
    return _impl(src, tgt, noise)
